# Initial kernel scaffold; baseline (speedup 1.0000x reference)
import jax
import jax.numpy as jnp
from jax.experimental import pallas as pl


def kernel(x, in_W, in_b, out_W, out_b, proc_W, proc_b, dest, jump_W, jump_b, ctx, phase, ptr_init):
    raise NotImplementedError("write your pallas kernel here")


# speedup vs baseline: 81.5082x; 81.5082x over previous
import functools
import jax
import jax.numpy as jnp
from jax import lax
from jax.experimental import pallas as pl
from jax.experimental.pallas import tpu as pltpu

B = 128
T = 32
P = 2048
D = 64
NB = 4
K = 2
TEMP = 8.0
E = T * NB
HALF = P / 2.0


def _ring_kernel(xT_ref, in_Wt_ref, in_b_ref, out_Wt_ref, out_b_ref,
                 proc_Wt_ref, proc_b_ref, destT_ref, jump_Wc_ref, jump_b_ref,
                 cs_ref, pb_ref, ptr0_ref,
                 y_ref,
                 SU, W5, BASE, PTR, HID):
    L = proc_Wt_ref.shape[0]
    PTR[...] = ptr0_ref[...]
    HID[...] = jnp.zeros_like(HID)
    SU[...] = jnp.zeros_like(SU)
    W5[...] = jnp.zeros_like(W5)
    BASE[...] = jnp.zeros_like(BASE)

    offs5 = lax.broadcasted_iota(jnp.int32, (5, B), 0).astype(jnp.float32) - K
    iotaE = lax.broadcasted_iota(jnp.int32, (E, 1), 0).astype(jnp.float32)
    iotaP = lax.broadcasted_iota(jnp.int32, (P, B), 0)

    def step_t(t, _):
        xt = xT_ref[t]
        inp = jnp.dot(in_Wt_ref[...], xt,
                      preferred_element_type=jnp.float32) + in_b_ref[...]
        acc = jnp.zeros((D, B), jnp.float32)
        for bi in range(NB):
            s_f = (t * NB + bi).astype(jnp.float32)
            ptr = PTR[bi][None, :]
            base_i = jnp.clip(jnp.floor(ptr).astype(jnp.int32), 0, P - 1)
            base_f = base_i.astype(jnp.float32)
            idx_f = jnp.mod(base_f + offs5, P)
            delta = jnp.remainder(idx_f - ptr + HALF, P) - HALF
            logits = -(delta * delta) / TEMP
            mx = jnp.max(logits, axis=0, keepdims=True)
            ex = jnp.exp(logits - mx)
            w = ex / jnp.sum(ex, axis=0, keepdims=True)

            dd = jnp.remainder(base_f - BASE[...] + HALF, P) - HALF
            valid = jnp.where(iotaE < s_f, 1.0, 0.0)
            c = jnp.zeros((E, B), jnp.float32)
            for jp in range(5):
                g = jnp.zeros((E, B), jnp.float32)
                for m in range(5):
                    g = g + jnp.where(dd == float(jp - m), w[m][None, :], 0.0)
                c = c + W5[jp] * g
            c = c * valid

            context = jnp.sum(c[:, None, :] * SU[...], axis=0)

            comb = inp + cs_ref[bi] * context + 0.1 * pb_ref[bi]
            su = jnp.tanh(comb + HID[bi])
            for l in range(L):
                su = jnp.tanh(jnp.dot(proc_Wt_ref[l], su,
                                      preferred_element_type=jnp.float32)
                              + proc_b_ref[l])
            HID[bi] = su
            SU[pl.ds(t * NB + bi, 1)] = su[None]
            W5[:, pl.ds(t * NB + bi, 1), :] = w[:, None, :]
            BASE[pl.ds(t * NB + bi, 1)] = base_f
            acc = acc + su

            jl = jnp.sum(jump_Wc_ref[bi] * su, axis=0, keepdims=True) \
                + jump_b_ref[bi]
            jd = jnp.where(jax.nn.sigmoid(jl) > 0.5, 1.0, 0.0)
            walk = jnp.remainder(ptr + 1.0, P)
            onehot = iotaP == base_i
            destv = jnp.sum(jnp.where(onehot, destT_ref[:, bi:bi + 1], 0.0),
                            axis=0, keepdims=True)
            PTR[bi] = jnp.remainder(jd * destv + (1.0 - jd) * walk, P)[0]
        y_ref[pl.ds(t, 1)] = (jnp.dot(out_Wt_ref[...], acc * (1.0 / NB),
                                      preferred_element_type=jnp.float32)
                              + out_b_ref[...])[None]
        return 0

    lax.fori_loop(0, T, step_t, 0)


@jax.jit
def kernel(x, in_W, in_b, out_W, out_b, proc_W, proc_b, dest, jump_W, jump_b,
           ctx, phase, ptr_init):
    xT = jnp.transpose(x, (1, 2, 0))
    in_Wt = jnp.transpose(in_W)
    out_Wt = jnp.transpose(out_W)
    proc_Wt = jnp.transpose(proc_W, (0, 2, 1))
    destT = jnp.transpose(dest)
    pb = jnp.concatenate(
        [phase, jnp.zeros((NB, D - phase.shape[1]), phase.dtype)], axis=1)
    yT = pl.pallas_call(
        _ring_kernel,
        out_shape=jax.ShapeDtypeStruct((T, 8, B), jnp.float32),
        scratch_shapes=[
            pltpu.VMEM((E, D, B), jnp.float32),
            pltpu.VMEM((5, E, B), jnp.float32),
            pltpu.VMEM((E, B), jnp.float32),
            pltpu.VMEM((NB, B), jnp.float32),
            pltpu.VMEM((NB, D, B), jnp.float32),
        ],
    )(xT, in_Wt, in_b[:, None], out_Wt, out_b[:, None],
      proc_Wt, proc_b[:, :, None], destT, jump_W[:, :, None],
      jump_b[:, None, None], jax.nn.sigmoid(ctx)[:, None, None],
      pb[:, :, None], ptr_init)
    return jnp.transpose(yT, (2, 0, 1))
